# baseline (speedup 1.0000x reference)
import functools

import jax
import jax.numpy as jnp
from jax import lax
from jax.experimental import pallas as pl
from jax.experimental.pallas import tpu as pltpu
from jax.experimental.pallas import tpu_sc as plsc

_KERNEL = 2

_NC = 2
_NS = 16
_LANES = 16


def _build_pool_kernel(group_sz, plane_sz, out_sz, planes_per_worker):
  mesh = plsc.VectorSubcoreMesh(
      core_axis_name="c", subcore_axis_name="s", num_cores=_NC,
      num_subcores=_NS)

  n_vec = out_sz // _LANES
  sc_per_group = _NS * planes_per_worker

  @functools.partial(
      pl.kernel,
      out_type=jax.ShapeDtypeStruct((_NC * sc_per_group * out_sz,),
                                    jnp.float32),
      mesh=mesh,
      compiler_params=pltpu.CompilerParams(
          needs_layout_passes=False, use_tc_tiling_on_sc=False),
      scratch_types=[
          pltpu.VMEM((out_sz,), jnp.int32),
          pltpu.VMEM((plane_sz,), jnp.float32),
          pltpu.VMEM((plane_sz,), jnp.float32),
          pltpu.VMEM((out_sz,), jnp.float32),
          pltpu.SemaphoreType.DMA,
          pltpu.SemaphoreType.DMA,
      ],
  )
  def pool_kernel(x_hbm, off_hbm, out_hbm, off_v, in0, in1, out_v, sem0,
                  sem1):
    c = lax.axis_index("c")
    s = lax.axis_index("s")
    in_base = c * group_sz + s * planes_per_worker
    out_base = c * sc_per_group + s * planes_per_worker
    pltpu.sync_copy(off_hbm.at[pl.ds(c * out_sz, out_sz)], off_v)

    ins = [in0, in1]
    sems = [sem0, sem1]
    copies = [None, None]
    copies[0] = pltpu.async_copy(
        x_hbm.at[pl.ds(in_base * plane_sz, plane_sz)], in0, sem0)
    for k in range(planes_per_worker):
      b = k & 1
      nb = (k + 1) & 1
      if k + 1 < planes_per_worker:
        copies[nb] = pltpu.async_copy(
            x_hbm.at[pl.ds((in_base + k + 1) * plane_sz, plane_sz)],
            ins[nb], sems[nb])
      copies[b].wait()
      in_buf = ins[b]

      batch = 8

      def _gather(vi, carry):
        vbase = vi * (batch * _LANES)
        ivs = [off_v[pl.ds(vbase + u * _LANES, _LANES)]
               for u in range(batch)]
        vals = [plsc.load_gather(in_buf, [iv]) for iv in ivs]
        for u in range(batch):
          out_v[pl.ds(vbase + u * _LANES, _LANES)] = vals[u]
        return carry

      lax.fori_loop(0, n_vec // batch, _gather, 0)

      pltpu.sync_copy(out_v,
                      out_hbm.at[pl.ds((out_base + k) * out_sz, out_sz)])

  return pool_kernel


def _build_tc_kernel(num_planes, group_sz, sc_per_group, out_h, out_w):
  tc_per_group = group_sz - sc_per_group
  n_tc = _NC * tc_per_group
  w = 2 * out_w

  def plane_of(i):
    return jnp.where(i < tc_per_group, sc_per_group + i,
                     group_sz + sc_per_group + (i - tc_per_group))

  def body(x_ref, m_ref, b_ref, o_ref):
    y = x_ref[0] * m_ref[0]
    yr = y.reshape(out_h, 2, w).sum(axis=1)
    o_ref[0] = jnp.dot(yr, b_ref[...],
                       preferred_element_type=jnp.float32,
                       precision=jax.lax.Precision.HIGHEST)

  return pl.pallas_call(
      body,
      grid=(n_tc,),
      in_specs=[
          pl.BlockSpec((1, 2 * out_h, w), lambda i: (plane_of(i), 0, 0)),
          pl.BlockSpec((1, 2 * out_h, w),
                       lambda i: (i // tc_per_group, 0, 0)),
          pl.BlockSpec((w, out_w), lambda i: (0, 0)),
      ],
      out_specs=pl.BlockSpec((1, out_h, out_w), lambda i: (i, 0, 0)),
      out_shape=jax.ShapeDtypeStruct((n_tc, out_h, out_w), jnp.float32),
  )


def kernel(x, T):
  B, C, H, W = x.shape
  k = _KERNEL
  out_h, out_w = H // k, W // k
  num_patch = out_h * out_w
  t_static = 4
  n_groups = B // t_static

  idx_key = jax.random.fold_in(jax.random.key(0), 1)
  sel = jax.random.randint(idx_key, (n_groups, 1, num_patch), 0, k * k)
  sel = sel[:, 0, :] + (jnp.asarray(T, sel.dtype) - t_static)
  sel = jnp.clip(sel, 0, k * k - 1).astype(jnp.int32)

  pp = jnp.arange(num_patch, dtype=jnp.int32)
  pi = pp // out_w
  pj = pp % out_w
  dh = sel // k
  dw = sel % k
  off = ((k * pi + dh) * W + (k * pj + dw)).astype(jnp.int32)

  num_planes = B * C
  group_sz = num_planes // n_groups

  planes_per_worker = 8
  sc_per_group = _NS * planes_per_worker
  tc_per_group = group_sz - sc_per_group

  pool = _build_pool_kernel(group_sz, H * W, num_patch, planes_per_worker)
  out_sc = pool(x.reshape(-1), off.reshape(-1)).reshape(
      _NC * sc_per_group, out_h, out_w)

  dh2 = dh.reshape(n_groups, out_h, out_w)
  dw2 = dw.reshape(n_groups, out_h, out_w)
  a_idx = jnp.arange(k, dtype=dh2.dtype)
  masks = (
      (dh2[:, :, None, :, None] == a_idx[None, None, :, None, None])
      & (dw2[:, :, None, :, None] == a_idx[None, None, None, None, :])
  ).astype(jnp.float32).reshape(n_groups, H, W)
  colsum = (jnp.arange(W)[:, None] // k ==
            jnp.arange(out_w)[None, :]).astype(jnp.float32)
  tck = _build_tc_kernel(num_planes, group_sz, sc_per_group, out_h, out_w)
  out_tc = tck(x.reshape(num_planes, H, W), masks, colsum)

  out = jnp.concatenate([
      out_sc[:sc_per_group], out_tc[:tc_per_group],
      out_sc[sc_per_group:], out_tc[tc_per_group:],
  ], axis=0)
  return out.reshape(B, C, out_h, out_w)


# speedup vs baseline: 1.4335x; 1.4335x over previous
import functools

import jax
import jax.numpy as jnp
from jax import lax
from jax.experimental import pallas as pl
from jax.experimental.pallas import tpu as pltpu
from jax.experimental.pallas import tpu_sc as plsc

_KERNEL = 2

_NC = 2
_NS = 16
_LANES = 16


def _build_pool_kernel(group_sz, plane_sz, out_sz, planes_per_worker):
  mesh = plsc.VectorSubcoreMesh(
      core_axis_name="c", subcore_axis_name="s", num_cores=_NC,
      num_subcores=_NS)

  n_vec = out_sz // _LANES
  sc_per_group = _NS * planes_per_worker

  @functools.partial(
      pl.kernel,
      out_type=jax.ShapeDtypeStruct((_NC * sc_per_group * out_sz,),
                                    jnp.float32),
      mesh=mesh,
      compiler_params=pltpu.CompilerParams(
          needs_layout_passes=False, use_tc_tiling_on_sc=False),
      scratch_types=[
          pltpu.VMEM((out_sz,), jnp.int32),
          pltpu.VMEM((plane_sz,), jnp.float32),
          pltpu.VMEM((plane_sz,), jnp.float32),
          pltpu.VMEM((out_sz,), jnp.float32),
          pltpu.SemaphoreType.DMA,
          pltpu.SemaphoreType.DMA,
      ],
  )
  def pool_kernel(x_hbm, off_hbm, out_hbm, off_v, in0, in1, out_v, sem0,
                  sem1):
    c = lax.axis_index("c")
    s = lax.axis_index("s")
    in_base = c * group_sz + s * planes_per_worker
    out_base = c * sc_per_group + s * planes_per_worker
    pltpu.sync_copy(off_hbm.at[pl.ds(c * out_sz, out_sz)], off_v)

    ins = [in0, in1]
    sems = [sem0, sem1]
    copies = [None, None]
    copies[0] = pltpu.async_copy(
        x_hbm.at[pl.ds(in_base * plane_sz, plane_sz)], in0, sem0)
    for k in range(planes_per_worker):
      b = k & 1
      nb = (k + 1) & 1
      if k + 1 < planes_per_worker:
        copies[nb] = pltpu.async_copy(
            x_hbm.at[pl.ds((in_base + k + 1) * plane_sz, plane_sz)],
            ins[nb], sems[nb])
      copies[b].wait()
      in_buf = ins[b]

      batch = 8

      def _gather(vi, carry):
        vbase = vi * (batch * _LANES)
        ivs = [off_v[pl.ds(vbase + u * _LANES, _LANES)]
               for u in range(batch)]
        vals = [plsc.load_gather(in_buf, [iv]) for iv in ivs]
        for u in range(batch):
          out_v[pl.ds(vbase + u * _LANES, _LANES)] = vals[u]
        return carry

      lax.fori_loop(0, n_vec // batch, _gather, 0)

      pltpu.sync_copy(out_v,
                      out_hbm.at[pl.ds((out_base + k) * out_sz, out_sz)])

  return pool_kernel


def _build_tc_kernel(num_planes, group_sz, sc_per_group, out_h, out_w):
  tc_per_group = group_sz - sc_per_group
  n_tc = _NC * tc_per_group
  w = 2 * out_w
  bp = 4
  blocks_per_group = tc_per_group // bp

  def block_of(i):
    return jnp.where(i < blocks_per_group,
                     sc_per_group // bp + i,
                     (group_sz + sc_per_group) // bp +
                     (i - blocks_per_group))

  def body(x_ref, m_ref, b_ref, o_ref):
    y = x_ref[...] * m_ref[...]
    yr = y.reshape(bp, out_h, 2, w).sum(axis=2)
    o_ref[...] = jax.lax.dot_general(
        yr, b_ref[...], (((2,), (0,)), ((), ())),
        preferred_element_type=jnp.float32)

  return pl.pallas_call(
      body,
      grid=(n_tc // bp,),
      in_specs=[
          pl.BlockSpec((bp, 2 * out_h, w), lambda i: (block_of(i), 0, 0)),
          pl.BlockSpec((1, 2 * out_h, w),
                       lambda i: (i // blocks_per_group, 0, 0)),
          pl.BlockSpec((w, out_w), lambda i: (0, 0)),
      ],
      out_specs=pl.BlockSpec((bp, out_h, out_w), lambda i: (i, 0, 0)),
      out_shape=jax.ShapeDtypeStruct((n_tc, out_h, out_w), jnp.float32),
  )


def kernel(x, T):
  B, C, H, W = x.shape
  k = _KERNEL
  out_h, out_w = H // k, W // k
  num_patch = out_h * out_w
  t_static = 4
  n_groups = B // t_static

  idx_key = jax.random.fold_in(jax.random.key(0), 1)
  sel = jax.random.randint(idx_key, (n_groups, 1, num_patch), 0, k * k)
  sel = sel[:, 0, :] + (jnp.asarray(T, sel.dtype) - t_static)
  sel = jnp.clip(sel, 0, k * k - 1).astype(jnp.int32)

  pp = jnp.arange(num_patch, dtype=jnp.int32)
  pi = pp // out_w
  pj = pp % out_w
  dh = sel // k
  dw = sel % k
  off = ((k * pi + dh) * W + (k * pj + dw)).astype(jnp.int32)

  num_planes = B * C
  group_sz = num_planes // n_groups

  planes_per_worker = 8
  sc_per_group = _NS * planes_per_worker
  tc_per_group = group_sz - sc_per_group

  pool = _build_pool_kernel(group_sz, H * W, num_patch, planes_per_worker)
  out_sc = pool(x.reshape(-1), off.reshape(-1)).reshape(
      _NC * sc_per_group, out_h, out_w)

  dh2 = dh.reshape(n_groups, out_h, out_w)
  dw2 = dw.reshape(n_groups, out_h, out_w)
  a_idx = jnp.arange(k, dtype=dh2.dtype)
  masks = (
      (dh2[:, :, None, :, None] == a_idx[None, None, :, None, None])
      & (dw2[:, :, None, :, None] == a_idx[None, None, None, None, :])
  ).astype(jnp.float32).reshape(n_groups, H, W)
  colsum = (jnp.arange(W)[:, None] // k ==
            jnp.arange(out_w)[None, :]).astype(jnp.float32)
  tck = _build_tc_kernel(num_planes, group_sz, sc_per_group, out_h, out_w)
  out_tc = tck(x.reshape(num_planes, H, W), masks, colsum)

  out = jnp.concatenate([
      out_sc[:sc_per_group], out_tc[:tc_per_group],
      out_sc[sc_per_group:], out_tc[tc_per_group:],
  ], axis=0)
  return out.reshape(B, C, out_h, out_w)


# speedup vs baseline: 2.1351x; 1.4895x over previous
import functools

import jax
import jax.numpy as jnp
from jax import lax
from jax.experimental import pallas as pl
from jax.experimental.pallas import tpu as pltpu
from jax.experimental.pallas import tpu_sc as plsc

_KERNEL = 2

_NC = 2
_NS = 16
_LANES = 16


def _build_pool_kernel(num_planes, plane_sz, out_sz, planes_per_worker):
  mesh = plsc.VectorSubcoreMesh(
      core_axis_name="c", subcore_axis_name="s", num_cores=_NC,
      num_subcores=_NS)

  n_vec = out_sz // _LANES

  @functools.partial(
      pl.kernel,
      out_type=jax.ShapeDtypeStruct((num_planes * out_sz,), jnp.float32),
      mesh=mesh,
      compiler_params=pltpu.CompilerParams(
          needs_layout_passes=False, use_tc_tiling_on_sc=False),
      scratch_types=[
          pltpu.VMEM((out_sz,), jnp.int32),
          pltpu.VMEM((plane_sz,), jnp.float32),
          pltpu.VMEM((plane_sz,), jnp.float32),
          pltpu.VMEM((out_sz,), jnp.float32),
          pltpu.SemaphoreType.DMA,
          pltpu.SemaphoreType.DMA,
      ],
  )
  def pool_kernel(x_hbm, off_hbm, out_hbm, off_v, in0, in1, out_v, sem0,
                  sem1):
    c = lax.axis_index("c")
    s = lax.axis_index("s")
    wid = c * _NS + s
    base = wid * planes_per_worker
    pltpu.sync_copy(off_hbm.at[pl.ds(c * out_sz, out_sz)], off_v)

    ins = [in0, in1]
    sems = [sem0, sem1]
    copies = [None, None]
    copies[0] = pltpu.async_copy(
        x_hbm.at[pl.ds(base * plane_sz, plane_sz)], in0, sem0)
    for k in range(planes_per_worker):
      b = k & 1
      nb = (k + 1) & 1
      if k + 1 < planes_per_worker:
        copies[nb] = pltpu.async_copy(
            x_hbm.at[pl.ds((base + k + 1) * plane_sz, plane_sz)], ins[nb],
            sems[nb])
      copies[b].wait()
      in_buf = ins[b]

      batch = 8

      def _gather(vi, carry):
        vbase = vi * (batch * _LANES)
        ivs = [off_v[pl.ds(vbase + u * _LANES, _LANES)]
               for u in range(batch)]
        vals = [plsc.load_gather(in_buf, [iv]) for iv in ivs]
        for u in range(batch):
          out_v[pl.ds(vbase + u * _LANES, _LANES)] = vals[u]
        return carry

      lax.fori_loop(0, n_vec // batch, _gather, 0)

      pltpu.sync_copy(out_v, out_hbm.at[pl.ds((base + k) * out_sz, out_sz)])

  return pool_kernel


def kernel(x, T):
  B, C, H, W = x.shape
  k = _KERNEL
  out_h, out_w = H // k, W // k
  num_patch = out_h * out_w
  t_static = 4
  n_groups = B // t_static

  idx_key = jax.random.fold_in(jax.random.key(0), 1)
  sel = jax.random.randint(idx_key, (n_groups, 1, num_patch), 0, k * k)
  sel = sel[:, 0, :] + (jnp.asarray(T, sel.dtype) - t_static)
  sel = jnp.clip(sel, 0, k * k - 1).astype(jnp.int32)

  pp = jnp.arange(num_patch, dtype=jnp.int32)
  pi = pp // out_w
  pj = pp % out_w
  dh = sel // k
  dw = sel % k
  off = ((k * pi + dh) * W + (k * pj + dw)).astype(jnp.int32)

  num_planes = B * C
  planes_per_worker = num_planes // (_NC * _NS)
  pool = _build_pool_kernel(num_planes, H * W, num_patch, planes_per_worker)
  out_flat = pool(x.reshape(-1), off.reshape(-1))
  return out_flat.reshape(B, C, out_h, out_w)
